# Initial kernel scaffold; baseline (speedup 1.0000x reference)
import jax
import jax.numpy as jnp
from jax.experimental import pallas as pl


def kernel(doc_embeds, edge_index, W1, b1, W2, b2):
    raise NotImplementedError("write your pallas kernel here")


# speedup vs baseline: 12.6588x; 12.6588x over previous
import functools

import jax
import jax.numpy as jnp
from jax import lax
from jax.experimental import pallas as pl
from jax.experimental.pallas import tpu as pltpu
from jax.experimental.pallas import tpu_sc as plsc

NC = 2
NS = 16
NW = NC * NS
CH = 128
NP = 10240
BM = 1024


def _fill(buf, n16, value):
    v = jnp.full((16,), value, jnp.float32)

    def body(i, _):
        buf[pl.ds(i * 16, 16)] = v
        return 0

    lax.fori_loop(0, n16, body, 0)


def _make_deg_kernel(ep):
    ew = ep // NW
    steps = ew // CH
    rt = NP // NS
    mesh = plsc.VectorSubcoreMesh(core_axis_name="c", subcore_axis_name="s")

    @functools.partial(
        pl.kernel,
        out_type=jax.ShapeDtypeStruct((NC * NP,), jnp.float32),
        mesh=mesh,
        scratch_types=[
            pltpu.VMEM((CH,), jnp.int32),
            pltpu.VMEM((CH,), jnp.float32),
            pltpu.VMEM((rt,), jnp.float32),
            pltpu.VMEM_SHARED((NP,), jnp.float32),
        ],
    )
    def deg_kernel(dst_hbm, out_hbm, didx, ones, zbuf, deg_sh):
        c = lax.axis_index("c")
        s = lax.axis_index("s")
        w = c * NS + s
        _fill(ones, CH // 16, 1.0)
        _fill(zbuf, rt // 16, 0.0)
        r0 = s * rt
        pltpu.sync_copy(zbuf, deg_sh.at[pl.ds(r0, rt)])
        plsc.subcore_barrier()

        def step(j, _):
            off = w * ew + j * CH
            pltpu.sync_copy(dst_hbm.at[pl.ds(off, CH)], didx)
            pltpu.sync_copy(ones, deg_sh.at[didx], add=True)
            return 0

        lax.fori_loop(0, steps, step, 0)
        plsc.subcore_barrier()
        pltpu.sync_copy(deg_sh.at[pl.ds(r0, rt)], out_hbm.at[pl.ds(c * NP + r0, rt)])

    return deg_kernel


def _make_agg_kernel(ep, d):
    ew = ep // NW
    steps = ew // CH
    rt = NP // NS
    mesh = plsc.VectorSubcoreMesh(core_axis_name="c", subcore_axis_name="s")

    @functools.partial(
        pl.kernel,
        out_type=jax.ShapeDtypeStruct((NC * NP, d), jnp.float32),
        mesh=mesh,
        scratch_types=[
            pltpu.VMEM((CH,), jnp.int32),
            pltpu.VMEM((CH,), jnp.int32),
            pltpu.VMEM((CH, d), jnp.float32),
            pltpu.VMEM((16, d), jnp.float32),
            pltpu.VMEM_SHARED((NP, d), jnp.float32),
            pltpu.SemaphoreType.DMA,
        ],
        compiler_params=pltpu.CompilerParams(use_tc_tiling_on_sc=False),
    )
    def agg(src_hbm, dst_hbm, y_hbm, out_hbm, sidx, didx, rows, zbuf, acc_sh, sem):
        c = lax.axis_index("c")
        s = lax.axis_index("s")
        w = c * NS + s
        zv = jnp.zeros((16,), jnp.float32)

        def zrow(i, _):
            for k in range(d // 16):
                zbuf[i, pl.ds(k * 16, 16)] = zv
            return 0

        lax.fori_loop(0, 16, zrow, 0)
        r0 = s * rt

        def zstep(t, _):
            pltpu.sync_copy(zbuf, acc_sh.at[pl.ds(r0 + t * 16, 16)])
            return 0

        lax.fori_loop(0, rt // 16, zstep, 0)
        plsc.subcore_barrier()

        def step(j, _):
            off = w * ew + j * CH
            pltpu.sync_copy(src_hbm.at[pl.ds(off, CH)], sidx)
            pltpu.sync_copy(dst_hbm.at[pl.ds(off, CH)], didx)
            pltpu.async_copy(y_hbm.at[sidx], rows, sem).wait()
            pltpu.sync_copy(rows, acc_sh.at[didx], add=True)
            return 0

        lax.fori_loop(0, steps, step, 0)
        plsc.subcore_barrier()
        pltpu.sync_copy(acc_sh.at[pl.ds(r0, rt)], out_hbm.at[pl.ds(c * NP + r0, rt)])

    return agg


def _tc_scale_matmul(xp, w1, d0, d1):
    din, dh = w1.shape
    g = NP // BM

    def body(x_ref, w_ref, d0_ref, d1_ref, y_ref, dv_ref):
        deg = d0_ref[...] + d1_ref[...] + 1.0
        dv = lax.rsqrt(deg)
        dv_ref[...] = dv
        y_ref[...] = jnp.dot(x_ref[...], w_ref[...], preferred_element_type=jnp.float32) * dv

    return pl.pallas_call(
        body,
        grid=(g,),
        in_specs=[
            pl.BlockSpec((BM, din), lambda i: (i, 0)),
            pl.BlockSpec((din, dh), lambda i: (0, 0)),
            pl.BlockSpec((BM, 1), lambda i: (i, 0)),
            pl.BlockSpec((BM, 1), lambda i: (i, 0)),
        ],
        out_specs=[
            pl.BlockSpec((BM, dh), lambda i: (i, 0)),
            pl.BlockSpec((BM, 1), lambda i: (i, 0)),
        ],
        out_shape=[
            jax.ShapeDtypeStruct((NP, dh), jnp.float32),
            jax.ShapeDtypeStruct((NP, 1), jnp.float32),
        ],
    )(xp, w1, d0, d1)


def _tc_mid(acc, y1, dv, b1, w2):
    dh, do = w2.shape
    g = NP // BM

    def body(a0_ref, a1_ref, y1_ref, dv_ref, b_ref, w_ref, y2_ref):
        dv = dv_ref[...]
        h = (a0_ref[...] + a1_ref[...] + y1_ref[...]) * dv + b_ref[...]
        h = jnp.maximum(h, 0.0)
        y2_ref[...] = jnp.dot(h, w_ref[...], preferred_element_type=jnp.float32) * dv

    return pl.pallas_call(
        body,
        grid=(g,),
        in_specs=[
            pl.BlockSpec((BM, dh), lambda i: (i, 0)),
            pl.BlockSpec((BM, dh), lambda i: (i + NP // BM, 0)),
            pl.BlockSpec((BM, dh), lambda i: (i, 0)),
            pl.BlockSpec((BM, 1), lambda i: (i, 0)),
            pl.BlockSpec((1, dh), lambda i: (0, 0)),
            pl.BlockSpec((dh, do), lambda i: (0, 0)),
        ],
        out_specs=pl.BlockSpec((BM, do), lambda i: (i, 0)),
        out_shape=jax.ShapeDtypeStruct((NP, do), jnp.float32),
    )(acc, acc, y1, dv, b1, w2)


def _tc_final(acc, y2, dv, b2):
    do = y2.shape[1]
    g = NP // BM

    def body(a0_ref, a1_ref, y2_ref, dv_ref, b_ref, o_ref):
        o_ref[...] = (a0_ref[...] + a1_ref[...] + y2_ref[...]) * dv_ref[...] + b_ref[...]

    return pl.pallas_call(
        body,
        grid=(g,),
        in_specs=[
            pl.BlockSpec((BM, do), lambda i: (i, 0)),
            pl.BlockSpec((BM, do), lambda i: (i + NP // BM, 0)),
            pl.BlockSpec((BM, do), lambda i: (i, 0)),
            pl.BlockSpec((BM, 1), lambda i: (i, 0)),
            pl.BlockSpec((1, do), lambda i: (0, 0)),
        ],
        out_specs=pl.BlockSpec((BM, do), lambda i: (i, 0)),
        out_shape=jax.ShapeDtypeStruct((NP, do), jnp.float32),
    )(acc, acc, y2, dv, b2)


@jax.jit
def kernel(doc_embeds, edge_index, W1, b1, W2, b2):
    n, din = doc_embeds.shape
    dh = W1.shape[1]
    do = W2.shape[1]
    e = edge_index.shape[1]
    chunk = NW * CH
    ep = ((e + chunk - 1) // chunk) * chunk

    src = edge_index[0].astype(jnp.int32)
    dst = edge_index[1].astype(jnp.int32)
    srcp = jnp.concatenate([src, jnp.zeros((ep - e,), jnp.int32)])
    dstp = jnp.concatenate([dst, jnp.full((ep - e,), n, jnp.int32)])
    xp = jnp.zeros((NP, din), jnp.float32).at[:n].set(doc_embeds)

    deg = _make_deg_kernel(ep)(dstp)
    d0 = deg[:NP].reshape(NP, 1)
    d1 = deg[NP:].reshape(NP, 1)

    y1, dv = _tc_scale_matmul(xp, W1, d0, d1)
    acc1 = _make_agg_kernel(ep, dh)(srcp, dstp, y1)
    y2 = _tc_mid(acc1, y1, dv, b1.reshape(1, dh), W2)
    acc2 = _make_agg_kernel(ep, do)(srcp, dstp, y2)
    out = _tc_final(acc2, y2, dv, b2.reshape(1, do))
    return out[:n]
